# baseline (speedup 1.0000x reference)
import functools

import jax
import jax.numpy as jnp
from jax import lax
from jax.experimental import pallas as pl
from jax.experimental.pallas import tpu as pltpu
from jax.experimental.pallas import tpu_sc as plsc

NUM_FIELDS = 26
ROWS_PER_FIELD = 100000
EMBED = 16
BATCH = 16384
N = BATCH * NUM_FIELDS

LANES = 16
NUM_CORES = 2
NUM_SUBCORES = 16
NW = NUM_CORES * NUM_SUBCORES
PER_W = N // NW
CHUNK = 416
NCHUNK = PER_W // CHUNK
VECS = CHUNK // LANES
NBUF = 8
NGROUP = NCHUNK // NBUF

REC_PER_B = 256
OUT_RECS = BATCH * REC_PER_B


def _sc_lookup(x_flat, table):
    mesh = plsc.VectorSubcoreMesh(core_axis_name="c", subcore_axis_name="s")

    @functools.partial(
        pl.kernel,
        mesh=mesh,
        out_type=jax.ShapeDtypeStruct((OUT_RECS, EMBED), jnp.float32),
        compiler_params=pltpu.CompilerParams(use_tc_tiling_on_sc=False),
        scratch_types=[
            pltpu.VMEM((NCHUNK, CHUNK), jnp.int32),
            pltpu.VMEM((NCHUNK, CHUNK), jnp.int32),
            pltpu.VMEM((NBUF, CHUNK, EMBED), jnp.float32),
            pltpu.SemaphoreType.DMA((NBUF,)),
            pltpu.SemaphoreType.DMA((NBUF,)),
        ],
    )
    def k(x_hbm, table_hbm, out_hbm, idx_v, oidx_v, rows_v, gsem, osem):
        wid = lax.axis_index("s") * NUM_CORES + lax.axis_index("c")
        base = wid * PER_W
        lane = lax.iota(jnp.int32, LANES)

        pltpu.sync_copy(x_hbm.at[wid], idx_v)

        def adjust(ci):
            row = idx_v.at[ci]
            orow = oidx_v.at[ci]

            def body(i, _):
                off = i * LANES
                p = base + ci * CHUNK + off + lane
                f = lax.rem(p, NUM_FIELDS)
                b = lax.div(p, NUM_FIELDS)
                row[pl.ds(off, LANES)] = (
                    row[pl.ds(off, LANES)] + f * ROWS_PER_FIELD
                )
                orow[pl.ds(off, LANES)] = b * REC_PER_B + f * 8
                return 0

            lax.fori_loop(0, VECS, body, 0)

        def start_gather(ci, b):
            pltpu.async_copy(
                table_hbm.at[idx_v.at[ci]], rows_v.at[b], gsem.at[b]
            )

        def wait_gather(b):
            pltpu.make_async_copy(
                out_hbm.at[pl.ds(0, CHUNK)], rows_v.at[b], gsem.at[b]
            ).wait()

        def start_out(ci, b):
            pltpu.async_copy(
                rows_v.at[b], out_hbm.at[oidx_v.at[ci]], osem.at[b]
            )

        def wait_out(b):
            pltpu.make_async_copy(
                rows_v.at[b], out_hbm.at[pl.ds(0, CHUNK)], osem.at[b]
            ).wait()

        def group(g, _):
            for b in range(NBUF):
                @pl.when(g > 0)
                def _():
                    wait_out(b)

                adjust(g * NBUF + b)
                start_gather(g * NBUF + b, b)
            for b in range(NBUF):
                wait_gather(b)
                start_out(g * NBUF + b, b)
            return 0

        lax.fori_loop(0, NGROUP, group, 0)

        for b in range(NBUF):
            wait_out(b)

    return k(x_flat, table)


def kernel(x, table):
    x_flat = x.reshape(NW, NCHUNK, CHUNK)
    out_pad = _sc_lookup(x_flat, table)
    return out_pad.reshape(BATCH, 32, 8, EMBED)[:, :NUM_FIELDS, 0, :]
